# baseline (speedup 1.0000x reference)
import jax
import jax.numpy as jnp
from jax.experimental import pallas as pl
from jax.experimental.pallas import tpu as pltpu

_NCHUNK = 8


def _dma_copy_body(x_hbm, o_hbm, sems):
    rows = x_hbm.shape[0]
    chunk = rows // _NCHUNK
    copies = [
        pltpu.make_async_copy(
            x_hbm.at[pl.ds(i * chunk, chunk)],
            o_hbm.at[pl.ds(i * chunk, chunk)],
            sems.at[i],
        )
        for i in range(_NCHUNK)
    ]
    for c in copies:
        c.start()
    for c in copies:
        c.wait()


def kernel(X):
    B, S, F = X.shape
    total = B * S * F
    rows = total // 128
    assert rows % _NCHUNK == 0

    flat = X.reshape(rows, 128)
    out = pl.pallas_call(
        _dma_copy_body,
        in_specs=[pl.BlockSpec(memory_space=pl.ANY)],
        out_specs=pl.BlockSpec(memory_space=pl.ANY),
        scratch_shapes=[pltpu.SemaphoreType.DMA((_NCHUNK,))],
        out_shape=jax.ShapeDtypeStruct((rows, 128), jnp.float32),
    )(flat)
    return out.reshape(B, S, F)


# speedup vs baseline: 7.6241x; 7.6241x over previous
import jax
import jax.numpy as jnp
from jax.experimental import pallas as pl


def _copy_body(x_ref, o_ref):
    o_ref[...] = x_ref[...]


def kernel(X):
    B, S, F = X.shape
    blk = 8
    grid = B // blk
    return pl.pallas_call(
        _copy_body,
        grid=(grid,),
        in_specs=[pl.BlockSpec((blk, S, F), lambda i: (i, 0, 0))],
        out_specs=pl.BlockSpec((blk, S, F), lambda i: (i, 0, 0)),
        out_shape=jax.ShapeDtypeStruct((B, S, F), jnp.float32),
    )(X)
